# baseline (device time: 27141 ns/iter reference)
import jax
import jax.numpy as jnp
from jax import lax
from jax.experimental import pallas as pl
from jax.experimental.pallas import tpu as pltpu

N_DEV = 16


def _gelu(y):
    c = 0.7978845608028654
    return 0.5 * y * (1.0 + jnp.tanh(c * (y + 0.044715 * y * y * y)))


def kernel(x, w_mat):
    m, k = x.shape
    _, n = w_mat.shape
    m_out = m // N_DEV
    mh = m // 2

    def body(x_ref, w_ref, out_ref, accbf_ref, comm_ref,
             send_sems, recv_sems):
        my = lax.axis_index("i")

        barrier = pltpu.get_barrier_semaphore()
        for b in range(N_DEV):
            @pl.when(b != my)
            def _():
                pl.semaphore_signal(
                    barrier, inc=1, device_id=(b,),
                    device_id_type=pl.DeviceIdType.MESH,
                )

        accbf_ref[:, :] = jnp.dot(
            x_ref[:, :].astype(jnp.bfloat16),
            w_ref[:, :].astype(jnp.bfloat16),
            preferred_element_type=jnp.float32,
        ).astype(jnp.bfloat16)

        pl.semaphore_wait(barrier, N_DEV - 1)

        def send(b):
            @pl.when(b != my)
            def _():
                q = jnp.mod(b - my - 1, N_DEV)
                rdma = pltpu.make_async_remote_copy(
                    src_ref=accbf_ref.at[pl.ds(b * m_out, m_out), :],
                    dst_ref=comm_ref.at[q],
                    send_sem=send_sems.at[b],
                    recv_sem=recv_sems.at[q],
                    device_id=(b,),
                    device_id_type=pl.DeviceIdType.MESH,
                )
                rdma.start()

        for b in range(N_DEV):
            send(b)

        def recv_wait(s):
            rdma = pltpu.make_async_remote_copy(
                src_ref=accbf_ref.at[pl.ds(0, m_out), :],
                dst_ref=comm_ref.at[s],
                send_sem=send_sems.at[0],
                recv_sem=recv_sems.at[s],
                device_id=(my,),
                device_id_type=pl.DeviceIdType.MESH,
            )
            rdma.wait_recv()

        myrow = pl.multiple_of(my * m_out, m_out)
        total = accbf_ref[pl.ds(myrow, m_out), :].astype(jnp.float32)
        for lo, hi in [(0, 4), (4, 8), (8, 12), (12, 14), (14, 15)]:
            for s in range(lo, hi):
                recv_wait(s)
            g = comm_ref[lo, :, :].astype(jnp.float32)
            for s in range(lo + 1, hi):
                g = g + comm_ref[s, :, :].astype(jnp.float32)
            total = total + g
        out_ref[:, :] = _gelu(total)

        for b in range(N_DEV):
            @pl.when(b != my)
            def _():
                rdma = pltpu.make_async_remote_copy(
                    src_ref=accbf_ref.at[pl.ds(b * m_out, m_out), :],
                    dst_ref=comm_ref.at[0],
                    send_sem=send_sems.at[b],
                    recv_sem=recv_sems.at[0],
                    device_id=(b,),
                    device_id_type=pl.DeviceIdType.MESH,
                )
                rdma.wait_send()

    return pl.pallas_call(
        body,
        out_shape=jax.ShapeDtypeStruct((m_out, n), jnp.float32),
        in_specs=[
            pl.BlockSpec(memory_space=pltpu.VMEM),
            pl.BlockSpec(memory_space=pltpu.VMEM),
        ],
        out_specs=pl.BlockSpec(memory_space=pltpu.VMEM),
        scratch_shapes=[
            pltpu.VMEM((m, n), jnp.bfloat16),
            pltpu.VMEM((N_DEV - 1, m_out, n), jnp.bfloat16),
            pltpu.SemaphoreType.DMA((N_DEV,)),
            pltpu.SemaphoreType.DMA((N_DEV - 1,)),
        ],
        compiler_params=pltpu.CompilerParams(collective_id=0),
    )(x, w_mat)
